# Initial kernel scaffold; baseline (speedup 1.0000x reference)
import jax
import jax.numpy as jnp
from jax.experimental import pallas as pl


def kernel(x, edge_index, W1, b1, W2, b2):
    raise NotImplementedError("write your pallas kernel here")


# speedup vs baseline: 13.1003x; 13.1003x over previous
import functools

import jax
import jax.numpy as jnp
from jax import lax
from jax.experimental import pallas as pl
from jax.experimental.pallas import tpu as pltpu
from jax.experimental.pallas import tpu_sc as plsc

N = 10000
E = 320000
D = 128

NC = 2
NS = 16
NW = NC * NS
EPW = E // NW
K = 80
NWIN = EPW // K

NPAD = 10240
ROWS_PER_TILE = NPAD // NS
ZCHUNK = 128
NZC = ROWS_PER_TILE // ZCHUNK


@functools.cache
def _mesh():
    return plsc.VectorSubcoreMesh(
        core_axis_name="c", subcore_axis_name="s",
        num_cores=NC, num_subcores=NS)


@functools.cache
def _sc_degree_call():
    return pl.kernel(
        _sc_degree_body,
        out_type=jax.ShapeDtypeStruct((NC * NPAD,), jnp.float32),
        mesh=_mesh(),
        scratch_types=[
            pltpu.VMEM((K,), jnp.int32),
            pltpu.VMEM((K,), jnp.float32),
            pltpu.VMEM((ROWS_PER_TILE,), jnp.float32),
            pltpu.VMEM_SHARED((NPAD,), jnp.float32),
        ],
    )


def _sc_degree_body(dst_hbm, z1_hbm, out_hbm, dstv, onesv, zb1, deg_sh):
    c = lax.axis_index("c")
    s = lax.axis_index("s")
    w = c * NS + s
    r0 = s * ROWS_PER_TILE

    pltpu.sync_copy(z1_hbm.at[pl.ds(r0, ROWS_PER_TILE)], zb1)
    pltpu.sync_copy(zb1, deg_sh.at[pl.ds(r0, ROWS_PER_TILE)])

    for j in range(K // 16):
        onesv[pl.ds(j * 16, 16)] = jnp.ones((16,), jnp.float32)
    plsc.subcore_barrier()

    def win(i, carry):
        base = w * EPW + i * K
        pltpu.sync_copy(dst_hbm.at[pl.ds(base, K)], dstv)
        pltpu.sync_copy(onesv, deg_sh.at[dstv], add=True)
        return carry

    lax.fori_loop(0, NWIN, win, 0)
    plsc.subcore_barrier()

    pltpu.sync_copy(deg_sh.at[pl.ds(r0, ROWS_PER_TILE)], zb1)
    pltpu.sync_copy(zb1, out_hbm.at[pl.ds(c * NPAD + r0, ROWS_PER_TILE)])


@functools.cache
def _sc_scatter_call():
    return pl.kernel(
        _sc_scatter_body,
        out_type=jax.ShapeDtypeStruct((NC, NPAD, D), jnp.float32),
        mesh=_mesh(),
        scratch_types=[
            pltpu.VMEM((K,), jnp.int32),
            pltpu.VMEM((K,), jnp.int32),
            pltpu.VMEM((K, D), jnp.float32),
            pltpu.VMEM((ZCHUNK, D), jnp.float32),
            pltpu.VMEM_SHARED((NPAD, D), jnp.float32),
            pltpu.SemaphoreType.DMA,
        ],
    )


def _sc_scatter_body(hs_hbm, src_hbm, dst_hbm, z2_hbm, out_hbm,
                     srcv, dstv, rowsv, zb, acc_sh, sem):
    c = lax.axis_index("c")
    s = lax.axis_index("s")
    w = c * NS + s

    for j in range(NZC):
        r0 = s * ROWS_PER_TILE + j * ZCHUNK
        pltpu.sync_copy(z2_hbm.at[pl.ds(r0, ZCHUNK)], zb)
        pltpu.sync_copy(zb, acc_sh.at[pl.ds(r0, ZCHUNK)])
    plsc.subcore_barrier()

    def win(i, carry):
        base = w * EPW + i * K
        pltpu.sync_copy(src_hbm.at[pl.ds(base, K)], srcv)
        pltpu.sync_copy(dst_hbm.at[pl.ds(base, K)], dstv)
        pltpu.async_copy(hs_hbm.at[srcv], rowsv, sem).wait()
        pltpu.sync_copy(rowsv, acc_sh.at[dstv], add=True)
        return carry

    lax.fori_loop(0, NWIN, win, 0)
    plsc.subcore_barrier()

    for j in range(NZC):
        r0 = s * ROWS_PER_TILE + j * ZCHUNK
        pltpu.sync_copy(acc_sh.at[pl.ds(r0, ZCHUNK)], zb)
        pltpu.sync_copy(zb, out_hbm.at[c, pl.ds(r0, ZCHUNK)])


def _tc1_body(deg2_ref, x_ref, w1_ref, hs_ref, dinv_ref):
    deg = 1.0 + deg2_ref[:, 0:1] + deg2_ref[:, 1:2]
    dinv = lax.rsqrt(deg)
    h = jnp.dot(x_ref[...], w1_ref[...], preferred_element_type=jnp.float32)
    hs_ref[...] = h * dinv
    dinv_ref[...] = dinv


def _tc2_body(dinv_ref, accp_ref, h1s_ref, b1_ref, w2_ref, h2s_ref):
    dinv = dinv_ref[...]
    acc = accp_ref[0, :N, :] + accp_ref[1, :N, :]
    z = dinv * (acc + h1s_ref[...]) + b1_ref[...]
    z = jnp.where(z > 0, z, jnp.exp(z) - 1.0)
    h2 = jnp.dot(z, w2_ref[...], preferred_element_type=jnp.float32)
    h2s_ref[...] = h2 * dinv


def _tc3_body(dinv_ref, accp_ref, h2s_ref, b2_ref, out_ref):
    dinv = dinv_ref[...]
    acc = accp_ref[0, :N, :] + accp_ref[1, :N, :]
    out_ref[...] = dinv * (acc + h2s_ref[...]) + b2_ref[...]


def _tc1(deg2, x, W1):
    return pl.pallas_call(
        _tc1_body,
        out_shape=(jax.ShapeDtypeStruct((N, D), jnp.float32),
                   jax.ShapeDtypeStruct((N, 1), jnp.float32)),
    )(deg2, x, W1)


def _tc2(dinv, accp, h1s, b1, W2):
    return pl.pallas_call(
        _tc2_body,
        out_shape=jax.ShapeDtypeStruct((N, D), jnp.float32),
    )(dinv, accp, h1s, b1, W2)


def _tc3(dinv, accp, h2s, b2):
    return pl.pallas_call(
        _tc3_body,
        out_shape=jax.ShapeDtypeStruct((N, D), jnp.float32),
    )(dinv, accp, h2s, b2)


def kernel(x, edge_index, W1, b1, W2, b2):
    src = edge_index[0].astype(jnp.int32)
    dst = edge_index[1].astype(jnp.int32)
    z1 = jnp.zeros((NPAD,), jnp.float32)
    z2 = jnp.zeros((NPAD, D), jnp.float32)

    degp = _sc_degree_call()(dst, z1)
    deg2 = jnp.transpose(degp.reshape(NC, NPAD)[:, :N])
    h1s, dinv = _tc1(deg2, x, W1)
    acc1 = _sc_scatter_call()(h1s, src, dst, z2)
    h2s = _tc2(dinv, acc1, h1s, b1.reshape(1, D), W2)
    acc2 = _sc_scatter_call()(h2s, src, dst, z2)
    return _tc3(dinv, acc2, h2s, b2.reshape(1, D))


# speedup vs baseline: 30.0224x; 2.2917x over previous
import functools

import jax
import jax.numpy as jnp
from jax import lax
from jax.experimental import pallas as pl
from jax.experimental.pallas import tpu as pltpu
from jax.experimental.pallas import tpu_sc as plsc

N = 10000
E = 320000
D = 128

NC = 2
NS = 16
NW = NC * NS

KD = 128
KW = 128
EPC = 10240
NWIN = EPC // KW
PHN = 4
PHE = EPC // PHN
PHW = PHE // KW
EPAD = NW * EPC

NPAD = 10240
ROWS_PER_TILE = NPAD // NS


@functools.cache
def _mesh():
    return plsc.VectorSubcoreMesh(
        core_axis_name="c", subcore_axis_name="s",
        num_cores=NC, num_subcores=NS)


@functools.cache
def _sc_degree_call():
    return pl.kernel(
        _sc_degree_body,
        out_type=jax.ShapeDtypeStruct((NC * NPAD,), jnp.float32),
        mesh=_mesh(),
        scratch_types=[
            pltpu.VMEM((EPC,), jnp.int32),
            pltpu.VMEM((KD,), jnp.float32),
            pltpu.VMEM((ROWS_PER_TILE,), jnp.float32),
            pltpu.VMEM_SHARED((NPAD,), jnp.float32),
            pltpu.SemaphoreType.DMA,
            pltpu.SemaphoreType.DMA,
        ],
    )


def _sc_degree_body(dst_hbm, z1_hbm, out_hbm, dstall, onesv, zb1, deg_sh,
                    sem, zsem):
    c = lax.axis_index("c")
    s = lax.axis_index("s")
    w = c * NS + s
    r0 = s * ROWS_PER_TILE

    pltpu.async_copy(dst_hbm.at[pl.ds(w * EPC, EPC)], dstall, sem)
    pltpu.sync_copy(z1_hbm.at[pl.ds(r0, ROWS_PER_TILE)], zb1)
    pltpu.sync_copy(zb1, deg_sh.at[pl.ds(r0, ROWS_PER_TILE)])
    for j in range(KD // 16):
        onesv[pl.ds(j * 16, 16)] = jnp.ones((16,), jnp.float32)
    pltpu.make_async_copy(dst_hbm.at[pl.ds(w * EPC, EPC)], dstall,
                          sem).wait()
    plsc.subcore_barrier()

    def chunk(m, carry):
        for j in range(8):
            i = m * 8 + j
            pltpu.async_copy(
                onesv, deg_sh.at[dstall.at[pl.ds(i * KD, KD)]], sem,
                add=True)
        for j in range(8):
            pltpu.make_async_copy(
                onesv, deg_sh.at[dstall.at[pl.ds(0, KD)]], sem).wait()
        return carry

    lax.fori_loop(0, EPC // KD // 8, chunk, 0)
    plsc.subcore_barrier()

    pltpu.sync_copy(deg_sh.at[pl.ds(r0, ROWS_PER_TILE)], zb1)
    pltpu.sync_copy(zb1, out_hbm.at[pl.ds(c * NPAD + r0, ROWS_PER_TILE)])


@functools.cache
def _sc_scatter_call():
    return pl.kernel(
        _sc_scatter_body,
        out_type=jax.ShapeDtypeStruct((NC, NPAD, D), jnp.float32),
        mesh=_mesh(),
        scratch_types=[
            pltpu.VMEM((PHE,), jnp.int32),
            pltpu.VMEM((PHE,), jnp.int32),
            pltpu.VMEM((PHE,), jnp.int32),
            pltpu.VMEM((PHE,), jnp.int32),
            pltpu.VMEM((KW, D), jnp.float32),
            pltpu.VMEM((KW, D), jnp.float32),
            pltpu.VMEM_SHARED((NPAD, D), jnp.float32),
            pltpu.SemaphoreType.DMA,
            pltpu.SemaphoreType.DMA,
            pltpu.SemaphoreType.DMA,
            pltpu.SemaphoreType.DMA,
            pltpu.SemaphoreType.DMA,
            pltpu.SemaphoreType.DMA,
            pltpu.SemaphoreType.DMA,
        ],
    )


def _sc_scatter_body(hs_hbm, src_hbm, dst_hbm, z2_hbm, out_hbm,
                     srcA, srcB, dstA, dstB, rows0, rows1, acc_sh,
                     isemA, isemB, zsem, g0, g1, s0, s1):
    c = lax.axis_index("c")
    s = lax.axis_index("s")
    w = c * NS + s
    r0 = s * ROWS_PER_TILE
    e0 = w * EPC

    srcb = (srcA, srcB)
    dstb = (dstA, dstB)
    isem = (isemA, isemB)
    rows = (rows0, rows1)
    gsem = (g0, g1)
    ssem = (s0, s1)

    def stage_idx(p):
        q = p % 2
        pltpu.async_copy(src_hbm.at[pl.ds(e0 + p * PHE, PHE)], srcb[q],
                         isem[q])
        pltpu.async_copy(dst_hbm.at[pl.ds(e0 + p * PHE, PHE)], dstb[q],
                         isem[q])

    def wait_idx(p):
        q = p % 2
        pltpu.make_async_copy(src_hbm.at[pl.ds(e0, PHE)], srcb[q],
                              isem[q]).wait()
        pltpu.make_async_copy(dst_hbm.at[pl.ds(e0, PHE)], dstb[q],
                              isem[q]).wait()

    stage_idx(0)
    stage_idx(1)
    pltpu.sync_copy(z2_hbm.at[pl.ds(r0, KW)], rows0)
    for j in range(ROWS_PER_TILE // KW):
        pltpu.async_copy(rows0, acc_sh.at[pl.ds(r0 + j * KW, KW)], zsem)
    for j in range(ROWS_PER_TILE // KW):
        pltpu.make_async_copy(
            rows0, acc_sh.at[pl.ds(r0, KW)], zsem).wait()
    wait_idx(0)
    plsc.subcore_barrier()

    def start_gather(q, gl, b):
        pltpu.async_copy(hs_hbm.at[srcb[q].at[pl.ds(gl * KW, KW)]], rows[b],
                         gsem[b])

    def wait_gather(b):
        pltpu.make_async_copy(hs_hbm.at[srcA.at[pl.ds(0, KW)]], rows[b],
                              gsem[b]).wait()

    def start_scatter(q, gl, b):
        pltpu.async_copy(rows[b], acc_sh.at[dstb[q].at[pl.ds(gl * KW, KW)]],
                         ssem[b], add=True)

    def wait_scatter(b):
        pltpu.make_async_copy(rows[b], acc_sh.at[dstA.at[pl.ds(0, KW)]],
                              ssem[b]).wait()

    start_gather(0, 0, 0)
    for p in range(PHN):
        q = p % 2

        wait_gather(0)
        start_scatter(q, 0, 0)
        if p > 0:
            wait_scatter(1)
            if p < PHN - 1:
                stage_idx(p + 1)
        start_gather(q, 1, 1)

        def pair(m, carry):
            for j in range(2):
                gl = 2 * m + 1 + j
                b = 1 - j
                wait_gather(b)
                start_scatter(q, gl, b)
                wait_scatter(1 - b)
                start_gather(q, gl + 1, 1 - b)
            return carry

        lax.fori_loop(0, (PHW - 2) // 2, pair, 0)

        wait_gather(1)
        start_scatter(q, PHW - 1, 1)
        wait_scatter(0)
        if p < PHN - 1:
            wait_idx(p + 1)
            start_gather(1 - q, 0, 0)

    wait_scatter(1)
    plsc.subcore_barrier()

    for j in range(ROWS_PER_TILE // KW):
        bb = j % 2
        if j >= 2:
            pltpu.make_async_copy(rows[bb], out_hbm.at[c, pl.ds(r0, KW)],
                                  gsem[bb]).wait()
        pltpu.sync_copy(acc_sh.at[pl.ds(r0 + j * KW, KW)], rows[bb])
        pltpu.async_copy(rows[bb], out_hbm.at[c, pl.ds(r0 + j * KW, KW)],
                         gsem[bb])
    for j in range(ROWS_PER_TILE // KW - 2, ROWS_PER_TILE // KW):
        pltpu.make_async_copy(rows[j % 2], out_hbm.at[c, pl.ds(r0, KW)],
                              gsem[j % 2]).wait()


def _tc1_body(deg2_ref, x_ref, w1_ref, hs_ref, dinv_ref):
    deg = 1.0 + deg2_ref[:, 0:1] + deg2_ref[:, 1:2]
    dinv = lax.rsqrt(deg)
    h = jnp.dot(x_ref[...], w1_ref[...], preferred_element_type=jnp.float32)
    hs_ref[...] = h * dinv
    dinv_ref[...] = dinv


def _tc2_body(dinv_ref, accp_ref, h1s_ref, b1_ref, w2_ref, h2s_ref):
    dinv = dinv_ref[...]
    acc = accp_ref[0, :N, :] + accp_ref[1, :N, :]
    z = dinv * (acc + h1s_ref[...]) + b1_ref[...]
    z = jnp.where(z > 0, z, jnp.exp(z) - 1.0)
    h2 = jnp.dot(z, w2_ref[...], preferred_element_type=jnp.float32)
    h2s_ref[...] = h2 * dinv


def _tc3_body(dinv_ref, accp_ref, h2s_ref, b2_ref, out_ref):
    dinv = dinv_ref[...]
    acc = accp_ref[0, :N, :] + accp_ref[1, :N, :]
    out_ref[...] = dinv * (acc + h2s_ref[...]) + b2_ref[...]


def _tc1(deg2, x, W1):
    return pl.pallas_call(
        _tc1_body,
        out_shape=(jax.ShapeDtypeStruct((N, D), jnp.float32),
                   jax.ShapeDtypeStruct((N, 1), jnp.float32)),
    )(deg2, x, W1)


def _tc2(dinv, accp, h1s, b1, W2):
    return pl.pallas_call(
        _tc2_body,
        out_shape=jax.ShapeDtypeStruct((N, D), jnp.float32),
    )(dinv, accp, h1s, b1, W2)


def _tc3(dinv, accp, h2s, b2):
    return pl.pallas_call(
        _tc3_body,
        out_shape=jax.ShapeDtypeStruct((N, D), jnp.float32),
    )(dinv, accp, h2s, b2)


def kernel(x, edge_index, W1, b1, W2, b2):
    src = edge_index[0].astype(jnp.int32)
    dst = edge_index[1].astype(jnp.int32)
    npadextra = EPAD - E
    src1 = jnp.concatenate(
        [src, jnp.arange(npadextra, dtype=jnp.int32) % N])
    dst1 = jnp.concatenate(
        [dst, N + (jnp.arange(npadextra, dtype=jnp.int32) % (NPAD - N))])
    z1 = jnp.zeros((NPAD,), jnp.float32)
    z2 = jnp.zeros((NPAD, D), jnp.float32)

    degp = _sc_degree_call()(dst1, z1)
    deg2 = jnp.transpose(degp.reshape(NC, NPAD)[:, :N])
    h1s, dinv = _tc1(deg2, x, W1)
    acc1 = _sc_scatter_call()(h1s, src1, dst1, z2)
    h2s = _tc2(dinv, acc1, h1s, b1.reshape(1, D), W2)
    acc2 = _sc_scatter_call()(h2s, src1, dst1, z2)
    return _tc3(dinv, acc2, h2s, b2.reshape(1, D))


# speedup vs baseline: 30.1232x; 1.0034x over previous
import functools

import jax
import jax.numpy as jnp
from jax import lax
from jax.experimental import pallas as pl
from jax.experimental.pallas import tpu as pltpu
from jax.experimental.pallas import tpu_sc as plsc

N = 10000
E = 320000
D = 128

NC = 2
NS = 16
NW = NC * NS

KD = 128
KW = 128
EPC = 10240
NWIN = EPC // KW
PHN = 5
PHE = EPC // PHN
PHW = PHE // KW
EPAD = NW * EPC

NPAD = 10240
ROWS_PER_TILE = NPAD // NS


@functools.cache
def _mesh():
    return plsc.VectorSubcoreMesh(
        core_axis_name="c", subcore_axis_name="s",
        num_cores=NC, num_subcores=NS)


@functools.cache
def _sc_degree_call():
    return pl.kernel(
        _sc_degree_body,
        out_type=jax.ShapeDtypeStruct((NC * NPAD,), jnp.float32),
        mesh=_mesh(),
        scratch_types=[
            pltpu.VMEM((EPC,), jnp.int32),
            pltpu.VMEM((KD,), jnp.float32),
            pltpu.VMEM((ROWS_PER_TILE,), jnp.float32),
            pltpu.VMEM_SHARED((NPAD,), jnp.float32),
            pltpu.SemaphoreType.DMA,
            pltpu.SemaphoreType.DMA,
        ],
    )


def _sc_degree_body(dst_hbm, z1_hbm, out_hbm, dstall, onesv, zb1, deg_sh,
                    sem, zsem):
    c = lax.axis_index("c")
    s = lax.axis_index("s")
    w = c * NS + s
    r0 = s * ROWS_PER_TILE

    pltpu.async_copy(dst_hbm.at[pl.ds(w * EPC, EPC)], dstall, sem)
    pltpu.sync_copy(z1_hbm.at[pl.ds(r0, ROWS_PER_TILE)], zb1)
    pltpu.sync_copy(zb1, deg_sh.at[pl.ds(r0, ROWS_PER_TILE)])
    for j in range(KD // 16):
        onesv[pl.ds(j * 16, 16)] = jnp.ones((16,), jnp.float32)
    pltpu.make_async_copy(dst_hbm.at[pl.ds(w * EPC, EPC)], dstall,
                          sem).wait()
    plsc.subcore_barrier()

    def chunk(m, carry):
        for j in range(8):
            i = m * 8 + j
            pltpu.async_copy(
                onesv, deg_sh.at[dstall.at[pl.ds(i * KD, KD)]], sem,
                add=True)
        for j in range(8):
            pltpu.make_async_copy(
                onesv, deg_sh.at[dstall.at[pl.ds(0, KD)]], sem).wait()
        return carry

    lax.fori_loop(0, EPC // KD // 8, chunk, 0)
    plsc.subcore_barrier()

    pltpu.sync_copy(deg_sh.at[pl.ds(r0, ROWS_PER_TILE)], zb1)
    pltpu.sync_copy(zb1, out_hbm.at[pl.ds(c * NPAD + r0, ROWS_PER_TILE)])


@functools.cache
def _sc_scatter_call():
    return pl.kernel(
        _sc_scatter_body,
        out_type=jax.ShapeDtypeStruct((NC, NPAD, D), jnp.float32),
        mesh=_mesh(),
        scratch_types=[
            pltpu.VMEM((PHW, KW), jnp.int32),
            pltpu.VMEM((PHW, KW), jnp.int32),
            pltpu.VMEM((PHW, KW), jnp.int32),
            pltpu.VMEM((PHW, KW), jnp.int32),
            pltpu.VMEM((KW, D), jnp.float32),
            pltpu.VMEM((KW, D), jnp.float32),
            pltpu.VMEM_SHARED((NPAD, D), jnp.float32),
            pltpu.SemaphoreType.DMA,
            pltpu.SemaphoreType.DMA,
            pltpu.SemaphoreType.DMA,
            pltpu.SemaphoreType.DMA,
            pltpu.SemaphoreType.DMA,
            pltpu.SemaphoreType.DMA,
            pltpu.SemaphoreType.DMA,
        ],
    )


def _sc_scatter_body(hs_hbm, src_hbm, dst_hbm, z2_hbm, out_hbm,
                     srcA, srcB, dstA, dstB, rows0, rows1, acc_sh,
                     isemA, isemB, zsem, g0, g1, s0, s1):
    c = lax.axis_index("c")
    s = lax.axis_index("s")
    w = c * NS + s
    r0 = s * ROWS_PER_TILE
    e0 = w * EPC

    srcb = (srcA, srcB)
    dstb = (dstA, dstB)
    isem = (isemA, isemB)
    rows = (rows0, rows1)
    gsem = (g0, g1)
    ssem = (s0, s1)

    row0 = w * NWIN

    def stage_idx(p):
        q = p % 2
        pltpu.async_copy(src_hbm.at[pl.ds(row0 + p * PHW, PHW)], srcb[q],
                         isem[q])
        pltpu.async_copy(dst_hbm.at[pl.ds(row0 + p * PHW, PHW)], dstb[q],
                         isem[q])

    def wait_idx(p):
        q = p % 2
        pltpu.make_async_copy(src_hbm.at[pl.ds(row0, PHW)], srcb[q],
                              isem[q]).wait()
        pltpu.make_async_copy(dst_hbm.at[pl.ds(row0, PHW)], dstb[q],
                              isem[q]).wait()

    stage_idx(0)
    stage_idx(1)
    pltpu.sync_copy(z2_hbm.at[pl.ds(r0, KW)], rows0)
    for j in range(ROWS_PER_TILE // KW):
        pltpu.async_copy(rows0, acc_sh.at[pl.ds(r0 + j * KW, KW)], zsem)
    for j in range(ROWS_PER_TILE // KW):
        pltpu.make_async_copy(
            rows0, acc_sh.at[pl.ds(r0, KW)], zsem).wait()
    wait_idx(0)
    plsc.subcore_barrier()

    def start_gather(q, gl, b):
        pltpu.async_copy(hs_hbm.at[srcb[q].at[gl]], rows[b], gsem[b])

    def wait_gather(b):
        pltpu.make_async_copy(hs_hbm.at[srcA.at[0]], rows[b],
                              gsem[b]).wait()

    def start_scatter(q, gl, b):
        pltpu.async_copy(rows[b], acc_sh.at[dstb[q].at[gl]], ssem[b],
                         add=True)

    def wait_scatter(b):
        pltpu.make_async_copy(rows[b], acc_sh.at[dstA.at[0]],
                              ssem[b]).wait()

    start_gather(0, 0, 0)
    for p in range(PHN):
        q = p % 2

        wait_gather(0)
        start_scatter(q, 0, 0)
        if p > 0:
            wait_scatter(1)
            if p < PHN - 1:
                stage_idx(p + 1)
        start_gather(q, 1, 1)

        def pair(m, carry):
            for j in range(2):
                gl = 2 * m + 1 + j
                b = 1 - j
                wait_gather(b)
                start_scatter(q, gl, b)
                wait_scatter(1 - b)
                start_gather(q, gl + 1, 1 - b)
            return carry

        lax.fori_loop(0, (PHW - 2) // 2, pair, 0)

        wait_gather(1)
        start_scatter(q, PHW - 1, 1)
        wait_scatter(0)
        if p < PHN - 1:
            wait_idx(p + 1)
            start_gather(1 - q, 0, 0)

    wait_scatter(1)
    plsc.subcore_barrier()

    for j in range(ROWS_PER_TILE // KW):
        bb = j % 2
        if j >= 2:
            pltpu.make_async_copy(rows[bb], out_hbm.at[c, pl.ds(r0, KW)],
                                  gsem[bb]).wait()
        pltpu.sync_copy(acc_sh.at[pl.ds(r0 + j * KW, KW)], rows[bb])
        pltpu.async_copy(rows[bb], out_hbm.at[c, pl.ds(r0 + j * KW, KW)],
                         gsem[bb])
    for j in range(ROWS_PER_TILE // KW - 2, ROWS_PER_TILE // KW):
        pltpu.make_async_copy(rows[j % 2], out_hbm.at[c, pl.ds(r0, KW)],
                              gsem[j % 2]).wait()


def _tc1_body(deg2_ref, x_ref, w1_ref, hs_ref, dinv_ref):
    deg = 1.0 + deg2_ref[:, 0:1] + deg2_ref[:, 1:2]
    dinv = lax.rsqrt(deg)
    h = jnp.dot(x_ref[...], w1_ref[...], preferred_element_type=jnp.float32)
    hs_ref[...] = h * dinv
    dinv_ref[...] = dinv


def _tc2_body(dinv_ref, accp_ref, h1s_ref, b1_ref, w2_ref, h2s_ref):
    dinv = dinv_ref[...]
    acc = accp_ref[0, :N, :] + accp_ref[1, :N, :]
    z = dinv * (acc + h1s_ref[...]) + b1_ref[...]
    z = jnp.where(z > 0, z, jnp.exp(z) - 1.0)
    h2 = jnp.dot(z, w2_ref[...], preferred_element_type=jnp.float32)
    h2s_ref[...] = h2 * dinv


def _tc3_body(dinv_ref, accp_ref, h2s_ref, b2_ref, out_ref):
    dinv = dinv_ref[...]
    acc = accp_ref[0, :N, :] + accp_ref[1, :N, :]
    out_ref[...] = dinv * (acc + h2s_ref[...]) + b2_ref[...]


def _tc1(deg2, x, W1):
    return pl.pallas_call(
        _tc1_body,
        out_shape=(jax.ShapeDtypeStruct((N, D), jnp.float32),
                   jax.ShapeDtypeStruct((N, 1), jnp.float32)),
    )(deg2, x, W1)


def _tc2(dinv, accp, h1s, b1, W2):
    return pl.pallas_call(
        _tc2_body,
        out_shape=jax.ShapeDtypeStruct((N, D), jnp.float32),
    )(dinv, accp, h1s, b1, W2)


def _tc3(dinv, accp, h2s, b2):
    return pl.pallas_call(
        _tc3_body,
        out_shape=jax.ShapeDtypeStruct((N, D), jnp.float32),
    )(dinv, accp, h2s, b2)


def kernel(x, edge_index, W1, b1, W2, b2):
    src = edge_index[0].astype(jnp.int32)
    dst = edge_index[1].astype(jnp.int32)
    npadextra = EPAD - E
    src1 = jnp.concatenate(
        [src, jnp.arange(npadextra, dtype=jnp.int32) % N])
    dst1 = jnp.concatenate(
        [dst, N + (jnp.arange(npadextra, dtype=jnp.int32) % (NPAD - N))])
    z1 = jnp.zeros((NPAD,), jnp.float32)
    z2 = jnp.zeros((NPAD, D), jnp.float32)

    src2 = src1.reshape(NW * NWIN, KW)
    dst2 = dst1.reshape(NW * NWIN, KW)

    degp = _sc_degree_call()(dst1, z1)
    deg2 = jnp.transpose(degp.reshape(NC, NPAD)[:, :N])
    h1s, dinv = _tc1(deg2, x, W1)
    acc1 = _sc_scatter_call()(h1s, src2, dst2, z2)
    h2s = _tc2(dinv, acc1, h1s, b1.reshape(1, D), W2)
    acc2 = _sc_scatter_call()(h2s, src2, dst2, z2)
    return _tc3(dinv, acc2, h2s, b2.reshape(1, D))


# speedup vs baseline: 34.0957x; 1.1319x over previous
import functools

import jax
import jax.numpy as jnp
from jax import lax
from jax.experimental import pallas as pl
from jax.experimental.pallas import tpu as pltpu
from jax.experimental.pallas import tpu_sc as plsc

N = 10000
E = 320000
D = 128

NC = 2
NS = 16
NW = NC * NS

KW = 128
WPW = 80
NPHASE = 2
WPP = WPW // NPHASE
EROWS = NW * WPW
EPAD = EROWS * KW

NPAD = 10240
ROWS_PER_TILE = NPAD // NS
ZCHUNK = 128
NZC = ROWS_PER_TILE // ZCHUNK


@functools.cache
def _mesh():
    return plsc.VectorSubcoreMesh(
        core_axis_name="c", subcore_axis_name="s",
        num_cores=NC, num_subcores=NS)


@functools.cache
def _sc_degree_call():
    return pl.kernel(
        _sc_degree_body,
        out_type=jax.ShapeDtypeStruct((NC * NPAD,), jnp.float32),
        mesh=_mesh(),
        scratch_types=[
            pltpu.VMEM((WPW, KW), jnp.int32),
            pltpu.VMEM((KW,), jnp.float32),
            pltpu.VMEM((ROWS_PER_TILE,), jnp.float32),
            pltpu.VMEM_SHARED((NPAD,), jnp.float32),
            pltpu.SemaphoreType.DMA,
        ],
    )


def _sc_degree_body(dst_hbm, z1_hbm, out_hbm, dstall, onesv, zb1, deg_sh, sem):
    c = lax.axis_index("c")
    s = lax.axis_index("s")
    w = c * NS + s
    r0 = s * ROWS_PER_TILE

    pltpu.sync_copy(z1_hbm.at[pl.ds(r0, ROWS_PER_TILE)], zb1)
    pltpu.sync_copy(zb1, deg_sh.at[pl.ds(r0, ROWS_PER_TILE)])
    pltpu.sync_copy(dst_hbm.at[pl.ds(w * WPW, WPW)], dstall)
    for j in range(KW // 16):
        onesv[pl.ds(j * 16, 16)] = jnp.ones((16,), jnp.float32)
    plsc.subcore_barrier()

    def chunk(m, carry):
        for j in range(8):
            pltpu.async_copy(onesv, deg_sh.at[dstall.at[m * 8 + j]], sem,
                             add=True)
        for j in range(8):
            pltpu.make_async_copy(onesv, deg_sh.at[dstall.at[0]], sem).wait()
        return carry

    lax.fori_loop(0, WPW // 8, chunk, 0)
    plsc.subcore_barrier()

    pltpu.sync_copy(deg_sh.at[pl.ds(r0, ROWS_PER_TILE)], zb1)
    pltpu.sync_copy(zb1, out_hbm.at[pl.ds(c * NPAD + r0, ROWS_PER_TILE)])


@functools.cache
def _sc_scatter_call():
    return pl.kernel(
        _sc_scatter_body,
        out_type=jax.ShapeDtypeStruct((NC, NPAD, D), jnp.float32),
        mesh=_mesh(),
        scratch_types=[
            pltpu.VMEM((WPP, KW), jnp.int32),
            pltpu.VMEM((WPP, KW), jnp.int32),
            pltpu.VMEM((KW, D), jnp.float32),
            pltpu.VMEM((KW, D), jnp.float32),
            pltpu.VMEM_SHARED((NPAD, D), jnp.float32),
            pltpu.SemaphoreType.DMA,
            pltpu.SemaphoreType.DMA,
            pltpu.SemaphoreType.DMA,
            pltpu.SemaphoreType.DMA,
        ],
    )


def _sc_scatter_body(hs_hbm, src_hbm, dst_hbm, z2_hbm, out_hbm,
                     srcall, dstall, rows0, rows1, acc_sh,
                     g0, g1, s0, s1):
    c = lax.axis_index("c")
    s = lax.axis_index("s")
    w = c * NS + s

    pltpu.sync_copy(z2_hbm.at[pl.ds(s * ROWS_PER_TILE, ZCHUNK)], rows0)
    for j in range(NZC):
        pltpu.async_copy(
            rows0, acc_sh.at[pl.ds(s * ROWS_PER_TILE + j * ZCHUNK, ZCHUNK)],
            g0)
    for j in range(NZC):
        pltpu.make_async_copy(
            rows0, acc_sh.at[pl.ds(0, ZCHUNK)], g0).wait()
    plsc.subcore_barrier()

    rows = (rows0, rows1)
    gsem = (g0, g1)
    ssem = (s0, s1)

    def start_gather(g, b):
        pltpu.async_copy(hs_hbm.at[srcall.at[g]], rows[b], gsem[b])

    def wait_gather(b):
        pltpu.make_async_copy(hs_hbm.at[srcall.at[0]], rows[b], gsem[b]).wait()

    def start_scatter(g, b):
        pltpu.async_copy(rows[b], acc_sh.at[dstall.at[g]], ssem[b], add=True)

    def wait_scatter(b):
        pltpu.make_async_copy(rows[b], acc_sh.at[dstall.at[0]], ssem[b]).wait()

    for p in range(NPHASE):
        row0 = w * WPW + p * WPP
        pltpu.sync_copy(src_hbm.at[pl.ds(row0, WPP)], srcall)
        pltpu.sync_copy(dst_hbm.at[pl.ds(row0, WPP)], dstall)

        start_gather(0, 0)
        start_gather(1, 1)
        wait_gather(0)
        start_scatter(0, 0)

        def pair(m, carry):
            g = 2 * m + 1
            wait_scatter(0)
            start_gather(g + 1, 0)
            wait_gather(1)
            start_scatter(g, 1)
            wait_scatter(1)
            start_gather(g + 2, 1)
            wait_gather(0)
            start_scatter(g + 1, 0)
            return carry

        lax.fori_loop(0, (WPP - 2) // 2, pair, 0)

        wait_gather(1)
        start_scatter(WPP - 1, 1)
        wait_scatter(0)
        wait_scatter(1)

    plsc.subcore_barrier()

    for j in range(NZC):
        bb = j % 2
        if j >= 2:
            pltpu.make_async_copy(rows[bb], out_hbm.at[c, pl.ds(0, ZCHUNK)],
                                  gsem[bb]).wait()
        r0 = s * ROWS_PER_TILE + j * ZCHUNK
        pltpu.sync_copy(acc_sh.at[pl.ds(r0, ZCHUNK)], rows[bb])
        pltpu.async_copy(rows[bb], out_hbm.at[c, pl.ds(r0, ZCHUNK)],
                         gsem[bb])
    for j in range(NZC - 2, NZC):
        pltpu.make_async_copy(rows[j % 2], out_hbm.at[c, pl.ds(0, ZCHUNK)],
                              gsem[j % 2]).wait()


def _tc1_body(deg2_ref, x_ref, w1_ref, hs_ref, dinv_ref):
    deg = 1.0 + deg2_ref[:, 0:1] + deg2_ref[:, 1:2]
    dinv = lax.rsqrt(deg)
    h = jnp.dot(x_ref[...], w1_ref[...], preferred_element_type=jnp.float32)
    hs_ref[...] = h * dinv
    dinv_ref[...] = dinv


def _tc2_body(dinv_ref, accp_ref, h1s_ref, b1_ref, w2_ref, h2s_ref):
    dinv = dinv_ref[...]
    acc = accp_ref[0, :N, :] + accp_ref[1, :N, :]
    z = dinv * (acc + h1s_ref[...]) + b1_ref[...]
    z = jnp.where(z > 0, z, jnp.exp(z) - 1.0)
    h2 = jnp.dot(z, w2_ref[...], preferred_element_type=jnp.float32)
    h2s_ref[...] = h2 * dinv


def _tc3_body(dinv_ref, accp_ref, h2s_ref, b2_ref, out_ref):
    dinv = dinv_ref[...]
    acc = accp_ref[0, :N, :] + accp_ref[1, :N, :]
    out_ref[...] = dinv * (acc + h2s_ref[...]) + b2_ref[...]


def _tc1(deg2, x, W1):
    return pl.pallas_call(
        _tc1_body,
        out_shape=(jax.ShapeDtypeStruct((N, D), jnp.float32),
                   jax.ShapeDtypeStruct((N, 1), jnp.float32)),
    )(deg2, x, W1)


def _tc2(dinv, accp, h1s, b1, W2):
    return pl.pallas_call(
        _tc2_body,
        out_shape=jax.ShapeDtypeStruct((N, D), jnp.float32),
    )(dinv, accp, h1s, b1, W2)


def _tc3(dinv, accp, h2s, b2):
    return pl.pallas_call(
        _tc3_body,
        out_shape=jax.ShapeDtypeStruct((N, D), jnp.float32),
    )(dinv, accp, h2s, b2)


def kernel(x, edge_index, W1, b1, W2, b2):
    src = edge_index[0].astype(jnp.int32)
    dst = edge_index[1].astype(jnp.int32)
    npadextra = EPAD - E
    src2 = jnp.concatenate(
        [src, jnp.arange(npadextra, dtype=jnp.int32) % N]).reshape(EROWS, KW)
    dst2 = jnp.concatenate(
        [dst, N + (jnp.arange(npadextra, dtype=jnp.int32) % (NPAD - N))]
    ).reshape(EROWS, KW)
    z1 = jnp.zeros((NPAD,), jnp.float32)
    z2 = jnp.zeros((NPAD, D), jnp.float32)

    degp = _sc_degree_call()(dst2, z1)
    deg2 = jnp.transpose(degp.reshape(NC, NPAD)[:, :N])
    h1s, dinv = _tc1(deg2, x, W1)
    acc1 = _sc_scatter_call()(h1s, src2, dst2, z2)
    h2s = _tc2(dinv, acc1, h1s, b1.reshape(1, D), W2)
    acc2 = _sc_scatter_call()(h2s, src2, dst2, z2)
    return _tc3(dinv, acc2, h2s, b2.reshape(1, D))
